# baseline (speedup 1.0000x reference)
import jax
import jax.numpy as jnp
import numpy as np
from jax.experimental import pallas as pl

B = 16
H = 224
W = 224
PATCH = 14
PH = H // PATCH
PW = W // PATCH
P = PH * PW
D = 1024
L = 2
NF = 4
NR = B - 1
NS = B // 2
KSEL = 5


def _pool_matrix() -> np.ndarray:
    a = np.zeros((PH, PH), np.float32)
    for i in range(PH):
        lo, hi = max(0, i - 1), min(PH - 1, i + 1)
        a[i, lo:hi + 1] = 1.0 / (hi - lo + 1)
    return np.kron(a, a).astype(np.float32)


def _resize_matrix() -> np.ndarray:
    scale = H / PH
    out = np.zeros((H, PH), np.float32)
    for i in range(H):
        x = (i + 0.5) / scale - 0.5
        w = np.maximum(0.0, 1.0 - np.abs(x - np.arange(PH)))
        out[i] = w / w.sum()
    return out


_POOL = _pool_matrix()
_RESIZE = _resize_matrix()


def _bf16_dot(x, y, dims):
    return jax.lax.dot_general(
        x.astype(jnp.bfloat16), y.astype(jnp.bfloat16), dims,
        preferred_element_type=jnp.float32)


def _k1_body(patches_ref, wp_ref, bp_ref, wl_ref, bl_ref, pool_ref, feats_ref):
    t = _bf16_dot(patches_ref[0], wp_ref[...],
                  (((1,), (0,)), ((), ()))) + bp_ref[...]
    pool = pool_ref[...]
    for l in range(L):
        x = _bf16_dot(t, wl_ref[l], (((1,), (0,)), ((), ()))) + bl_ref[l]
        f = jax.nn.gelu(x)
        feats_ref[l, 0] = f
        feats_ref[2 + l, 0] = _bf16_dot(pool, f, (((1,), (0,)), ((), ())))


def _k2_body(q_ref, r_ref, ma_ref, mb_ref):
    q = q_ref[0, 0]
    r = r_ref[0, 0]
    s = _bf16_dot(q, r, (((1,), (1,)), ((), ())))
    sqq = jnp.sum(q * q, axis=1)
    sqr = jnp.sum(r * r, axis=1)
    rowmin = sqq + jnp.min(sqr[None, :] - 2.0 * s, axis=1)
    colmin = sqr + jnp.min(sqq[:, None] - 2.0 * s, axis=0)
    ma_ref[0, 0, 0] = jnp.sqrt(jnp.maximum(rowmin, 1e-12))
    mb_ref[0, 0, 0] = jnp.sqrt(jnp.maximum(colmin, 1e-12))


def _k3a_body(ma_ref, mb_ref, scores_ref):
    ma = ma_ref[...]
    mb = mb_ref[...]
    img = jax.lax.broadcasted_iota(jnp.int32, ma.shape, 3) // P
    rnd = jax.lax.broadcasted_iota(jnp.int32, ma.shape, 1)
    side = (img - rnd) % NR
    is_a = (img == B - 1) | ((side >= 1) & (side <= NS - 1))
    vals = jnp.where(is_a, ma, mb)
    total = jnp.zeros((NF, 1, 1, B * P), jnp.float32)
    big = jnp.float32(3.0e38)
    for _ in range(KSEL):
        mv = jnp.min(vals, axis=1, keepdims=True)
        total = total + mv
        eq = vals <= mv
        idx = jnp.where(eq, rnd, NR)
        first = rnd == jnp.min(idx, axis=1, keepdims=True)
        vals = jnp.where(first, big, vals)
    scores_ref[...] = jnp.mean(total, axis=0)[0] * (1.0 / KSEL)


def _k3b_body(scores_ref, m_ref, pix_ref, final_ref):
    g = scores_ref[0]
    final_ref[0, 0] = jnp.full((128,), jnp.max(g), jnp.float32)
    m = m_ref[...]
    a1 = jax.lax.dot_general(m, g, (((1,), (0,)), ((), ())),
                             preferred_element_type=jnp.float32)
    pix_ref[0] = jax.lax.dot_general(a1, m, (((1,), (1,)), ((), ())),
                                     preferred_element_type=jnp.float32)


@jax.jit
def kernel(pixel_values, W_patch, b_patch, W_layers, b_layers):
    patches = pixel_values.reshape(B, 3, PH, PATCH, PW, PATCH)
    patches = patches.transpose(0, 2, 4, 1, 3, 5).reshape(B, P, 3 * PATCH * PATCH)
    cdim = patches.shape[-1]

    feats = pl.pallas_call(
        _k1_body,
        grid=(B,),
        in_specs=[
            pl.BlockSpec((1, P, cdim), lambda b: (b, 0, 0)),
            pl.BlockSpec((cdim, D), lambda b: (0, 0)),
            pl.BlockSpec((D,), lambda b: (0,)),
            pl.BlockSpec((L, D, D), lambda b: (0, 0, 0)),
            pl.BlockSpec((L, D), lambda b: (0, 0)),
            pl.BlockSpec((P, P), lambda b: (0, 0)),
        ],
        out_specs=pl.BlockSpec((NF, 1, P, D), lambda b: (0, b, 0, 0)),
        out_shape=jax.ShapeDtypeStruct((NF, B, P, D), jnp.float32),
    )(patches, W_patch, b_patch, W_layers, b_layers, jnp.asarray(_POOL))

    def _a_idx(r, s):
        return jnp.where(s == 0, B - 1, (r + s) % NR)

    def _b_idx(r, s):
        return (r - s) % NR

    ma, mb = pl.pallas_call(
        _k2_body,
        grid=(NF, NR, NS),
        in_specs=[
            pl.BlockSpec((1, 1, P, D), lambda f, r, s: (f, _a_idx(r, s), 0, 0)),
            pl.BlockSpec((1, 1, P, D), lambda f, r, s: (f, _b_idx(r, s), 0, 0)),
        ],
        out_specs=[
            pl.BlockSpec((1, 1, 1, P), lambda f, r, s: (f, r, 0, _a_idx(r, s))),
            pl.BlockSpec((1, 1, 1, P), lambda f, r, s: (f, r, 0, _b_idx(r, s))),
        ],
        out_shape=[
            jax.ShapeDtypeStruct((NF, NR, 1, B * P), jnp.float32),
            jax.ShapeDtypeStruct((NF, NR, 1, B * P), jnp.float32),
        ],
    )(feats, feats)

    scores = pl.pallas_call(
        _k3a_body,
        in_specs=[
            pl.BlockSpec((NF, NR, 1, B * P), lambda: (0, 0, 0, 0)),
            pl.BlockSpec((NF, NR, 1, B * P), lambda: (0, 0, 0, 0)),
        ],
        out_specs=pl.BlockSpec((1, B * P), lambda: (0, 0)),
        out_shape=jax.ShapeDtypeStruct((1, B * P), jnp.float32),
    )(ma, mb)

    pix, final = pl.pallas_call(
        _k3b_body,
        grid=(B,),
        in_specs=[
            pl.BlockSpec((1, PH, PW), lambda b: (b, 0, 0)),
            pl.BlockSpec((H, PH), lambda b: (0, 0)),
        ],
        out_specs=[
            pl.BlockSpec((1, H, W), lambda b: (b, 0, 0)),
            pl.BlockSpec((1, 1, 128), lambda b: (b, 0, 0)),
        ],
        out_shape=[
            jax.ShapeDtypeStruct((B, H, W), jnp.float32),
            jax.ShapeDtypeStruct((B, 1, 128), jnp.float32),
        ],
    )(scores.reshape(B, PH, PW), jnp.asarray(_RESIZE))

    return final[:, 0, 0], pix


# speedup vs baseline: 12.1273x; 12.1273x over previous
import jax
import jax.numpy as jnp
import numpy as np
from jax.experimental import pallas as pl

B = 16
H = 224
W = 224
PATCH = 14
PH = H // PATCH
PW = W // PATCH
P = PH * PW
D = 1024
L = 2
NF = 4
NR = B - 1
NS = B // 2
KSEL = 5


def _pool_matrix() -> np.ndarray:
    a = np.zeros((PH, PH), np.float32)
    for i in range(PH):
        lo, hi = max(0, i - 1), min(PH - 1, i + 1)
        a[i, lo:hi + 1] = 1.0 / (hi - lo + 1)
    return np.kron(a, a).astype(np.float32)


def _resize_matrix() -> np.ndarray:
    scale = H / PH
    out = np.zeros((H, PH), np.float32)
    for i in range(H):
        x = (i + 0.5) / scale - 0.5
        w = np.maximum(0.0, 1.0 - np.abs(x - np.arange(PH)))
        out[i] = w / w.sum()
    return out


_POOL = _pool_matrix()
_RESIZE = _resize_matrix()


def _bf16_dot(x, y, dims):
    return jax.lax.dot_general(
        x.astype(jnp.bfloat16), y.astype(jnp.bfloat16), dims,
        preferred_element_type=jnp.float32)


def _k1_body(patches_ref, wp_ref, bp_ref, wl_ref, bl_ref, pool_ref,
             feats_ref, sqb_ref, sql_ref):
    t = _bf16_dot(patches_ref[0], wp_ref[...],
                  (((1,), (0,)), ((), ()))) + bp_ref[...]
    pool = pool_ref[...]
    ones = jnp.ones((D, 128), jnp.bfloat16)

    def emit(i, f):
        feats_ref[i, 0] = f.astype(jnp.bfloat16)
        f2 = f * f
        hi = f2.astype(jnp.bfloat16)
        lo = (f2 - hi.astype(jnp.float32)).astype(jnp.bfloat16)
        dims = (((1,), (0,)), ((), ()))
        sqb_ref[i, 0] = (
            jax.lax.dot_general(hi, ones, dims, preferred_element_type=jnp.float32)
            + jax.lax.dot_general(lo, ones, dims, preferred_element_type=jnp.float32))
        sql_ref[i, 0] = jnp.sum(f2, axis=1)

    for l in range(L):
        x = _bf16_dot(t, wl_ref[l], (((1,), (0,)), ((), ()))) + bl_ref[l]
        f = jax.nn.gelu(x)
        emit(l, f)
        emit(2 + l, _bf16_dot(pool, f, (((1,), (0,)), ((), ()))))


def _k2_body(q_ref, r_ref, sqa_ref, sqb_ref, ma_ref, mb_ref):
    qa = q_ref[0, 0]
    rb = r_ref[0, 0]
    dims = (((1,), (1,)), ((), ()))
    st = jax.lax.dot_general(rb, qa, dims, preferred_element_type=jnp.float32)
    s2 = jax.lax.dot_general(qa, rb, dims, preferred_element_type=jnp.float32)
    sqa = sqa_ref[0, 0]
    sqb = sqb_ref[0, 0]
    sqa2 = jnp.concatenate([sqa, sqa], axis=1)
    sqb2 = jnp.concatenate([sqb, sqb], axis=1)
    ma_ref[0, 0, 0] = jnp.min(sqb2 - 2.0 * st, axis=0)
    mb_ref[0, 0, 0] = jnp.min(sqa2 - 2.0 * s2, axis=0)


def _k3a_body(ma_ref, mb_ref, sql_ref, scores_ref):
    ma = ma_ref[...]
    mb = mb_ref[...]
    sql = sql_ref[...][:, None]
    img = jax.lax.broadcasted_iota(jnp.int32, ma.shape, 3) // P
    rnd = jax.lax.broadcasted_iota(jnp.int32, ma.shape, 1)
    side = (img - rnd) % NR
    is_a = (img == B - 1) | ((side >= 1) & (side <= NS - 1))
    vals = jnp.where(is_a, ma, mb)
    total = jnp.zeros((NF, 1, 1, B * P), jnp.float32)
    big = jnp.float32(3.0e38)
    for _ in range(KSEL):
        mv = jnp.min(vals, axis=1, keepdims=True)
        total = total + jnp.sqrt(jnp.maximum(mv + sql, 1e-12))
        eq = vals <= mv
        idx = jnp.where(eq, rnd, NR)
        first = rnd == jnp.min(idx, axis=1, keepdims=True)
        vals = jnp.where(first, big, vals)
    scores_ref[...] = jnp.mean(total, axis=0)[0] * (1.0 / KSEL)


def _k3b_body(scores_ref, m_ref, pix_ref, final_ref):
    g = scores_ref[0]
    final_ref[0, 0] = jnp.full((128,), jnp.max(g), jnp.float32)
    m = m_ref[...]
    a1 = jax.lax.dot_general(m, g, (((1,), (0,)), ((), ())),
                             preferred_element_type=jnp.float32)
    pix_ref[0] = jax.lax.dot_general(a1, m, (((1,), (1,)), ((), ())),
                                     preferred_element_type=jnp.float32)


@jax.jit
def kernel(pixel_values, W_patch, b_patch, W_layers, b_layers):
    patches = pixel_values.reshape(B, 3, PH, PATCH, PW, PATCH)
    patches = patches.transpose(0, 2, 4, 1, 3, 5).reshape(B, P, 3 * PATCH * PATCH)
    cdim = patches.shape[-1]

    feats, sqb, sql = pl.pallas_call(
        _k1_body,
        grid=(B,),
        in_specs=[
            pl.BlockSpec((1, P, cdim), lambda b: (b, 0, 0)),
            pl.BlockSpec((cdim, D), lambda b: (0, 0)),
            pl.BlockSpec((D,), lambda b: (0,)),
            pl.BlockSpec((L, D, D), lambda b: (0, 0, 0)),
            pl.BlockSpec((L, D), lambda b: (0, 0)),
            pl.BlockSpec((P, P), lambda b: (0, 0)),
        ],
        out_specs=[
            pl.BlockSpec((NF, 1, P, D), lambda b: (0, b, 0, 0)),
            pl.BlockSpec((NF, 1, P, 128), lambda b: (0, b, 0, 0)),
            pl.BlockSpec((NF, 1, P), lambda b: (0, 0, b)),
        ],
        out_shape=[
            jax.ShapeDtypeStruct((NF, B, P, D), jnp.bfloat16),
            jax.ShapeDtypeStruct((NF, B, P, 128), jnp.float32),
            jax.ShapeDtypeStruct((NF, 1, B * P), jnp.float32),
        ],
    )(patches, W_patch, b_patch, W_layers, b_layers, jnp.asarray(_POOL))

    def _a_idx(r, s):
        return jnp.where(s == 0, B - 1, (r + s) % NR)

    def _b_idx(r, s):
        return (r - s) % NR

    ma, mb = pl.pallas_call(
        _k2_body,
        grid=(NF, NR, NS),
        in_specs=[
            pl.BlockSpec((1, 1, P, D), lambda f, r, s: (f, _a_idx(r, s), 0, 0)),
            pl.BlockSpec((1, 1, P, D), lambda f, r, s: (f, _b_idx(r, s), 0, 0)),
            pl.BlockSpec((1, 1, P, 128), lambda f, r, s: (f, _a_idx(r, s), 0, 0)),
            pl.BlockSpec((1, 1, P, 128), lambda f, r, s: (f, _b_idx(r, s), 0, 0)),
        ],
        out_specs=[
            pl.BlockSpec((1, 1, 1, P), lambda f, r, s: (f, r, 0, _a_idx(r, s))),
            pl.BlockSpec((1, 1, 1, P), lambda f, r, s: (f, r, 0, _b_idx(r, s))),
        ],
        out_shape=[
            jax.ShapeDtypeStruct((NF, NR, 1, B * P), jnp.float32),
            jax.ShapeDtypeStruct((NF, NR, 1, B * P), jnp.float32),
        ],
    )(feats, feats, sqb, sqb)

    scores = pl.pallas_call(
        _k3a_body,
        in_specs=[
            pl.BlockSpec((NF, NR, 1, B * P), lambda: (0, 0, 0, 0)),
            pl.BlockSpec((NF, NR, 1, B * P), lambda: (0, 0, 0, 0)),
            pl.BlockSpec((NF, 1, B * P), lambda: (0, 0, 0)),
        ],
        out_specs=pl.BlockSpec((1, B * P), lambda: (0, 0)),
        out_shape=jax.ShapeDtypeStruct((1, B * P), jnp.float32),
    )(ma, mb, sql)

    pix, final = pl.pallas_call(
        _k3b_body,
        grid=(B,),
        in_specs=[
            pl.BlockSpec((1, PH, PW), lambda b: (b, 0, 0)),
            pl.BlockSpec((H, PH), lambda b: (0, 0)),
        ],
        out_specs=[
            pl.BlockSpec((1, H, W), lambda b: (b, 0, 0)),
            pl.BlockSpec((1, 1, 128), lambda b: (b, 0, 0)),
        ],
        out_shape=[
            jax.ShapeDtypeStruct((B, H, W), jnp.float32),
            jax.ShapeDtypeStruct((B, 1, 128), jnp.float32),
        ],
    )(scores.reshape(B, PH, PW), jnp.asarray(_RESIZE))

    return final[:, 0, 0], pix


# speedup vs baseline: 14.8642x; 1.2257x over previous
import jax
import jax.numpy as jnp
import numpy as np
from jax.experimental import pallas as pl

B = 16
H = 224
W = 224
PATCH = 14
PH = H // PATCH
PW = W // PATCH
P = PH * PW
D = 1024
L = 2
NF = 4
NR = B - 1
NS = B // 2
KSEL = 5


def _pool_matrix() -> np.ndarray:
    a = np.zeros((PH, PH), np.float32)
    for i in range(PH):
        lo, hi = max(0, i - 1), min(PH - 1, i + 1)
        a[i, lo:hi + 1] = 1.0 / (hi - lo + 1)
    return np.kron(a, a).astype(np.float32)


def _resize_matrix() -> np.ndarray:
    scale = H / PH
    out = np.zeros((H, PH), np.float32)
    for i in range(H):
        x = (i + 0.5) / scale - 0.5
        w = np.maximum(0.0, 1.0 - np.abs(x - np.arange(PH)))
        out[i] = w / w.sum()
    return out


_POOL = _pool_matrix()
_RESIZE = _resize_matrix()


def _bf16_dot(x, y, dims):
    return jax.lax.dot_general(
        x.astype(jnp.bfloat16), y.astype(jnp.bfloat16), dims,
        preferred_element_type=jnp.float32)


def _k1_body(patches_ref, wp_ref, bp_ref, wl_ref, bl_ref, pool_ref,
             feats_ref, sqb_ref, sql_ref):
    t = _bf16_dot(patches_ref[0], wp_ref[...],
                  (((1,), (0,)), ((), ()))) + bp_ref[...]
    pool = pool_ref[...]
    ones = jnp.ones((D, 128), jnp.bfloat16)

    def emit(i, f):
        feats_ref[i, 0] = f.astype(jnp.bfloat16)
        f2 = f * f
        hi = f2.astype(jnp.bfloat16)
        lo = (f2 - hi.astype(jnp.float32)).astype(jnp.bfloat16)
        dims = (((1,), (0,)), ((), ()))
        sqb_ref[i, 0] = (
            jax.lax.dot_general(hi, ones, dims, preferred_element_type=jnp.float32)
            + jax.lax.dot_general(lo, ones, dims, preferred_element_type=jnp.float32))
        sql_ref[i, 0] = jnp.sum(f2, axis=1)

    for l in range(L):
        x = _bf16_dot(t, wl_ref[l], (((1,), (0,)), ((), ()))) + bl_ref[l]
        f = jax.nn.gelu(x)
        emit(l, f)
        emit(2 + l, _bf16_dot(pool, f, (((1,), (0,)), ((), ()))))


def _k2_body(feats_ref, sqb_all_ref, ma_ref, mb_ref):
    r = pl.program_id(1)
    s = pl.program_id(2)
    a = jnp.where(s == 0, B - 1, (r + s) % NR)
    b = (r - s) % NR
    qa = feats_ref[0, a]
    rb = feats_ref[0, b]
    dims = (((1,), (1,)), ((), ()))
    st = jax.lax.dot_general(rb, qa, dims, preferred_element_type=jnp.float32)
    s2 = jax.lax.dot_general(qa, rb, dims, preferred_element_type=jnp.float32)
    sqa = sqb_all_ref[0, a]
    sqb = sqb_all_ref[0, b]
    sqa2 = jnp.concatenate([sqa, sqa], axis=1)
    sqb2 = jnp.concatenate([sqb, sqb], axis=1)
    ma_ref[0, 0, 0] = jnp.min(sqb2 - 2.0 * st, axis=0)
    mb_ref[0, 0, 0] = jnp.min(sqa2 - 2.0 * s2, axis=0)


def _k3a_body(ma_ref, mb_ref, sql_ref, scores_ref):
    ma = ma_ref[...]
    mb = mb_ref[...]
    sql = sql_ref[...][:, None]
    img = jax.lax.broadcasted_iota(jnp.int32, ma.shape, 3) // P
    rnd = jax.lax.broadcasted_iota(jnp.int32, ma.shape, 1)
    side = (img - rnd) % NR
    is_a = (img == B - 1) | ((side >= 1) & (side <= NS - 1))
    vals = jnp.where(is_a, ma, mb)
    total = jnp.zeros((NF, 1, 1, B * P), jnp.float32)
    big = jnp.float32(3.0e38)
    for _ in range(KSEL):
        mv = jnp.min(vals, axis=1, keepdims=True)
        total = total + jnp.sqrt(jnp.maximum(mv + sql, 1e-12))
        eq = vals <= mv
        idx = jnp.where(eq, rnd, NR)
        first = rnd == jnp.min(idx, axis=1, keepdims=True)
        vals = jnp.where(first, big, vals)
    scores_ref[...] = jnp.mean(total, axis=0)[0] * (1.0 / KSEL)


def _k3b_body(scores_ref, m_ref, pix_ref, final_ref):
    g = scores_ref[0]
    final_ref[0, 0] = jnp.full((128,), jnp.max(g), jnp.float32)
    m = m_ref[...]
    a1 = jax.lax.dot_general(m, g, (((1,), (0,)), ((), ())),
                             preferred_element_type=jnp.float32)
    pix_ref[0] = jax.lax.dot_general(a1, m, (((1,), (1,)), ((), ())),
                                     preferred_element_type=jnp.float32)


@jax.jit
def kernel(pixel_values, W_patch, b_patch, W_layers, b_layers):
    patches = pixel_values.reshape(B, 3, PH, PATCH, PW, PATCH)
    patches = patches.transpose(0, 2, 4, 1, 3, 5).reshape(B, P, 3 * PATCH * PATCH)
    cdim = patches.shape[-1]

    feats, sqb, sql = pl.pallas_call(
        _k1_body,
        grid=(B,),
        in_specs=[
            pl.BlockSpec((1, P, cdim), lambda b: (b, 0, 0)),
            pl.BlockSpec((cdim, D), lambda b: (0, 0)),
            pl.BlockSpec((D,), lambda b: (0,)),
            pl.BlockSpec((L, D, D), lambda b: (0, 0, 0)),
            pl.BlockSpec((L, D), lambda b: (0, 0)),
            pl.BlockSpec((P, P), lambda b: (0, 0)),
        ],
        out_specs=[
            pl.BlockSpec((NF, 1, P, D), lambda b: (0, b, 0, 0)),
            pl.BlockSpec((NF, 1, P, 128), lambda b: (0, b, 0, 0)),
            pl.BlockSpec((NF, 1, P), lambda b: (0, 0, b)),
        ],
        out_shape=[
            jax.ShapeDtypeStruct((NF, B, P, D), jnp.bfloat16),
            jax.ShapeDtypeStruct((NF, B, P, 128), jnp.float32),
            jax.ShapeDtypeStruct((NF, 1, B * P), jnp.float32),
        ],
    )(patches, W_patch, b_patch, W_layers, b_layers, jnp.asarray(_POOL))

    def _a_idx(r, s):
        return jnp.where(s == 0, B - 1, (r + s) % NR)

    def _b_idx(r, s):
        return (r - s) % NR

    ma, mb = pl.pallas_call(
        _k2_body,
        grid=(NF, NR, NS),
        in_specs=[
            pl.BlockSpec((1, B, P, D), lambda f, r, s: (f, 0, 0, 0)),
            pl.BlockSpec((1, B, P, 128), lambda f, r, s: (f, 0, 0, 0)),
        ],
        out_specs=[
            pl.BlockSpec((1, 1, 1, P), lambda f, r, s: (f, r, 0, _a_idx(r, s))),
            pl.BlockSpec((1, 1, 1, P), lambda f, r, s: (f, r, 0, _b_idx(r, s))),
        ],
        out_shape=[
            jax.ShapeDtypeStruct((NF, NR, 1, B * P), jnp.float32),
            jax.ShapeDtypeStruct((NF, NR, 1, B * P), jnp.float32),
        ],
    )(feats, sqb)

    scores = pl.pallas_call(
        _k3a_body,
        in_specs=[
            pl.BlockSpec((NF, NR, 1, B * P), lambda: (0, 0, 0, 0)),
            pl.BlockSpec((NF, NR, 1, B * P), lambda: (0, 0, 0, 0)),
            pl.BlockSpec((NF, 1, B * P), lambda: (0, 0, 0)),
        ],
        out_specs=pl.BlockSpec((1, B * P), lambda: (0, 0)),
        out_shape=jax.ShapeDtypeStruct((1, B * P), jnp.float32),
    )(ma, mb, sql)

    pix, final = pl.pallas_call(
        _k3b_body,
        grid=(B,),
        in_specs=[
            pl.BlockSpec((1, PH, PW), lambda b: (b, 0, 0)),
            pl.BlockSpec((H, PH), lambda b: (0, 0)),
        ],
        out_specs=[
            pl.BlockSpec((1, H, W), lambda b: (b, 0, 0)),
            pl.BlockSpec((1, 1, 128), lambda b: (b, 0, 0)),
        ],
        out_shape=[
            jax.ShapeDtypeStruct((B, H, W), jnp.float32),
            jax.ShapeDtypeStruct((B, 1, 128), jnp.float32),
        ],
    )(scores.reshape(B, PH, PW), jnp.asarray(_RESIZE))

    return final[:, 0, 0], pix


# speedup vs baseline: 17.7608x; 1.1949x over previous
import jax
import jax.numpy as jnp
import numpy as np
from jax.experimental import pallas as pl

B = 16
H = 224
W = 224
PATCH = 14
PH = H // PATCH
PW = W // PATCH
P = PH * PW
D = 1024
L = 2
NF = 4
NN = B // 2
NRN = NN - 1
NRT = NRN + 2
KSEL = 5


def _pool_matrix() -> np.ndarray:
    a = np.zeros((PH, PH), np.float32)
    for i in range(PH):
        lo, hi = max(0, i - 1), min(PH - 1, i + 1)
        a[i, lo:hi + 1] = 1.0 / (hi - lo + 1)
    return np.kron(a, a).astype(np.float32)


def _resize_matrix() -> np.ndarray:
    scale = H / PH
    out = np.zeros((H, PH), np.float32)
    for i in range(H):
        x = (i + 0.5) / scale - 0.5
        w = np.maximum(0.0, 1.0 - np.abs(x - np.arange(PH)))
        out[i] = w / w.sum()
    return out


_POOL = _pool_matrix()
_RESIZE = _resize_matrix()


def _bf16_dot(x, y, dims):
    return jax.lax.dot_general(
        x.astype(jnp.bfloat16), y.astype(jnp.bfloat16), dims,
        preferred_element_type=jnp.float32)


def _k1_body(patches_ref, wp_ref, bp_ref, wl_ref, bl_ref, pool_ref,
             feats_ref, sqb_ref, sql_ref):
    t = _bf16_dot(patches_ref[0], wp_ref[...],
                  (((1,), (0,)), ((), ()))) + bp_ref[...]
    pool = pool_ref[...]
    ones = jnp.ones((D, 128), jnp.bfloat16)

    def emit(i, f):
        feats_ref[i, 0] = f.astype(jnp.bfloat16)
        f2 = f * f
        hi = f2.astype(jnp.bfloat16)
        lo = (f2 - hi.astype(jnp.float32)).astype(jnp.bfloat16)
        dims = (((1,), (0,)), ((), ()))
        sqb_ref[i, 0] = (
            jax.lax.dot_general(hi, ones, dims, preferred_element_type=jnp.float32)
            + jax.lax.dot_general(lo, ones, dims, preferred_element_type=jnp.float32))
        sql_ref[i, 0] = jnp.sum(f2, axis=1)

    for l in range(L):
        x = _bf16_dot(t, wl_ref[l], (((1,), (0,)), ((), ()))) + bl_ref[l]
        f = jax.nn.gelu(x)
        emit(l, f)
        emit(2 + l, _bf16_dot(pool, f, (((1,), (0,)), ((), ()))))


def _node_pair(rnd, s):
    diag_n = (rnd - NRN) * (NN // 2) + s
    na = jnp.where(rnd >= NRN, diag_n,
                   jnp.where(s == 0, NN - 1, (rnd + s) % NRN))
    nb = jnp.where(rnd >= NRN, diag_n, (rnd - s) % NRN)
    return na, nb


def _k2_body(feats_ref, sqb_all_ref, ma_ref, mb_ref):
    rnd = pl.program_id(1)
    s = pl.program_id(2)
    na, nb = _node_pair(rnd, s)
    qa = feats_ref[0, pl.ds(na * 2, 2)].reshape(2 * P, D)
    rb = feats_ref[0, pl.ds(nb * 2, 2)].reshape(2 * P, D)
    dims = (((1,), (1,)), ((), ()))
    sqa = sqb_all_ref[0, pl.ds(na * 2, 2)].reshape(2 * P, 128)
    sqb = sqb_all_ref[0, pl.ds(nb * 2, 2)].reshape(2 * P, 128)
    st = jax.lax.dot_general(rb, qa, dims, preferred_element_type=jnp.float32)
    sqb4 = jnp.concatenate([sqb, sqb, sqb, sqb], axis=1)
    ma_ref[0, 0] = jnp.min((sqb4 - 2.0 * st).reshape(2, P, 2 * P), axis=1)

    @pl.when(rnd < NRN)
    def _cross():
        s2 = jax.lax.dot_general(qa, rb, dims,
                                 preferred_element_type=jnp.float32)
        sqa4 = jnp.concatenate([sqa, sqa, sqa, sqa], axis=1)
        mb_ref[0, 0] = jnp.min((sqa4 - 2.0 * s2).reshape(2, P, 2 * P), axis=1)


def _k3a_body(ma_ref, mb_ref, sql_ref, scores_ref):
    ma = ma_ref[...]
    mb = mb_ref[...]
    sql = sql_ref[...][:, None]
    big = jnp.float32(3.0e38)
    img = jax.lax.broadcasted_iota(jnp.int32, ma.shape, 3) // P
    rnd = jax.lax.broadcasted_iota(jnp.int32, ma.shape, 1)
    opp = jax.lax.broadcasted_iota(jnp.int32, ma.shape, 2)
    node = img // 2
    side = (node - rnd) % NRN
    is_a = (node == NN - 1) | ((side >= 1) & (side <= NN // 2 - 1))
    cross = jnp.where(is_a, ma, mb)
    diag_ok = (rnd == NRN + node // (NN // 2)) & (opp != img % 2)
    vals = jnp.where(rnd < NRN, cross, jnp.where(diag_ok, ma, big))
    cidx = rnd * 2 + opp
    total = jnp.zeros((NF, 1, 1, B * P), jnp.float32)
    for _ in range(KSEL):
        mv = jnp.min(vals, axis=(1, 2), keepdims=True)
        total = total + jnp.sqrt(jnp.maximum(mv + sql, 1e-12))
        eq = vals <= mv
        idx = jnp.where(eq, cidx, 2 * NRT)
        first = cidx == jnp.min(idx, axis=(1, 2), keepdims=True)
        vals = jnp.where(first, big, vals)
    scores_ref[...] = jnp.mean(total, axis=0)[0, 0][None] * (1.0 / KSEL)


def _k3b_body(scores_ref, m_ref, pix_ref, final_ref):
    g = scores_ref[0]
    final_ref[0, 0] = jnp.full((128,), jnp.max(g), jnp.float32)
    m = m_ref[...]
    a1 = jax.lax.dot_general(m, g, (((1,), (0,)), ((), ())),
                             preferred_element_type=jnp.float32)
    pix_ref[0] = jax.lax.dot_general(a1, m, (((1,), (1,)), ((), ())),
                                     preferred_element_type=jnp.float32)


@jax.jit
def kernel(pixel_values, W_patch, b_patch, W_layers, b_layers):
    patches = pixel_values.reshape(B, 3, PH, PATCH, PW, PATCH)
    patches = patches.transpose(0, 2, 4, 1, 3, 5).reshape(B, P, 3 * PATCH * PATCH)
    cdim = patches.shape[-1]

    feats, sqb, sql = pl.pallas_call(
        _k1_body,
        grid=(B,),
        in_specs=[
            pl.BlockSpec((1, P, cdim), lambda b: (b, 0, 0)),
            pl.BlockSpec((cdim, D), lambda b: (0, 0)),
            pl.BlockSpec((D,), lambda b: (0,)),
            pl.BlockSpec((L, D, D), lambda b: (0, 0, 0)),
            pl.BlockSpec((L, D), lambda b: (0, 0)),
            pl.BlockSpec((P, P), lambda b: (0, 0)),
        ],
        out_specs=[
            pl.BlockSpec((NF, 1, P, D), lambda b: (0, b, 0, 0)),
            pl.BlockSpec((NF, 1, P, 128), lambda b: (0, b, 0, 0)),
            pl.BlockSpec((NF, 1, P), lambda b: (0, 0, b)),
        ],
        out_shape=[
            jax.ShapeDtypeStruct((NF, B, P, D), jnp.bfloat16),
            jax.ShapeDtypeStruct((NF, B, P, 128), jnp.float32),
            jax.ShapeDtypeStruct((NF, 1, B * P), jnp.float32),
        ],
    )(patches, W_patch, b_patch, W_layers, b_layers, jnp.asarray(_POOL))

    ma, mb = pl.pallas_call(
        _k2_body,
        grid=(NF, NRT, NN // 2),
        in_specs=[
            pl.BlockSpec((1, B, P, D), lambda f, r, s: (f, 0, 0, 0)),
            pl.BlockSpec((1, B, P, 128), lambda f, r, s: (f, 0, 0, 0)),
        ],
        out_specs=[
            pl.BlockSpec((1, 1, 2, 2 * P),
                         lambda f, r, s: (f, r, 0, _node_pair(r, s)[0])),
            pl.BlockSpec((1, 1, 2, 2 * P),
                         lambda f, r, s: (f, r, 0, _node_pair(r, s)[1])),
        ],
        out_shape=[
            jax.ShapeDtypeStruct((NF, NRT, 2, B * P), jnp.float32),
            jax.ShapeDtypeStruct((NF, NRT, 2, B * P), jnp.float32),
        ],
    )(feats, sqb)

    scores = pl.pallas_call(
        _k3a_body,
        in_specs=[
            pl.BlockSpec((NF, NRT, 2, B * P), lambda: (0, 0, 0, 0)),
            pl.BlockSpec((NF, NRT, 2, B * P), lambda: (0, 0, 0, 0)),
            pl.BlockSpec((NF, 1, B * P), lambda: (0, 0, 0)),
        ],
        out_specs=pl.BlockSpec((1, B * P), lambda: (0, 0)),
        out_shape=jax.ShapeDtypeStruct((1, B * P), jnp.float32),
    )(ma, mb, sql)

    pix, final = pl.pallas_call(
        _k3b_body,
        grid=(B,),
        in_specs=[
            pl.BlockSpec((1, PH, PW), lambda b: (b, 0, 0)),
            pl.BlockSpec((H, PH), lambda b: (0, 0)),
        ],
        out_specs=[
            pl.BlockSpec((1, H, W), lambda b: (b, 0, 0)),
            pl.BlockSpec((1, 1, 128), lambda b: (b, 0, 0)),
        ],
        out_shape=[
            jax.ShapeDtypeStruct((B, H, W), jnp.float32),
            jax.ShapeDtypeStruct((B, 1, 128), jnp.float32),
        ],
    )(scores.reshape(B, PH, PW), jnp.asarray(_RESIZE))

    return final[:, 0, 0], pix


# speedup vs baseline: 20.7002x; 1.1655x over previous
import jax
import jax.numpy as jnp
import numpy as np
from jax.experimental import pallas as pl

B = 16
H = 224
W = 224
PATCH = 14
PH = H // PATCH
PW = W // PATCH
P = PH * PW
D = 1024
L = 2
NF = 4
NG = 4
NN = B // NG
NRN = NN - 1
NSL = NN // 2
NRT = NRN + 2
KSEL = 5


def _pool_matrix() -> np.ndarray:
    a = np.zeros((PH, PH), np.float32)
    for i in range(PH):
        lo, hi = max(0, i - 1), min(PH - 1, i + 1)
        a[i, lo:hi + 1] = 1.0 / (hi - lo + 1)
    return np.kron(a, a).astype(np.float32)


def _resize_matrix() -> np.ndarray:
    scale = H / PH
    out = np.zeros((H, PH), np.float32)
    for i in range(H):
        x = (i + 0.5) / scale - 0.5
        w = np.maximum(0.0, 1.0 - np.abs(x - np.arange(PH)))
        out[i] = w / w.sum()
    return out


_POOL = _pool_matrix()
_RESIZE = _resize_matrix()


def _bf16_dot(x, y, dims):
    return jax.lax.dot_general(
        x.astype(jnp.bfloat16), y.astype(jnp.bfloat16), dims,
        preferred_element_type=jnp.float32)


def _k1_body(patches_ref, wp_ref, bp_ref, wl_ref, bl_ref, pool_ref,
             feats_ref, sqb_ref, sql_ref):
    t = _bf16_dot(patches_ref[0], wp_ref[...],
                  (((1,), (0,)), ((), ()))) + bp_ref[...]
    pool = pool_ref[...]
    ones = jnp.ones((D, 128), jnp.bfloat16)

    def emit(i, f):
        feats_ref[i, 0] = f.astype(jnp.bfloat16)
        f2 = f * f
        hi = f2.astype(jnp.bfloat16)
        lo = (f2 - hi.astype(jnp.float32)).astype(jnp.bfloat16)
        dims = (((1,), (0,)), ((), ()))
        sqb_ref[i, 0] = (
            jax.lax.dot_general(hi, ones, dims, preferred_element_type=jnp.float32)
            + jax.lax.dot_general(lo, ones, dims, preferred_element_type=jnp.float32))
        sql_ref[i, 0] = jnp.sum(f2, axis=1)

    for l in range(L):
        x = _bf16_dot(t, wl_ref[l], (((1,), (0,)), ((), ()))) + bl_ref[l]
        f = jax.nn.gelu(x)
        emit(l, f)
        emit(2 + l, _bf16_dot(pool, f, (((1,), (0,)), ((), ()))))


def _node_pair(rnd, s):
    diag_n = (rnd - NRN) * NSL + s
    na = jnp.where(rnd >= NRN, diag_n,
                   jnp.where(s == 0, NN - 1, (rnd + s) % NRN))
    nb = jnp.where(rnd >= NRN, diag_n, (rnd - s) % NRN)
    return na, nb


def _k2_body(feats_ref, sqb_all_ref, ma_ref, mb_ref):
    rnd = pl.program_id(1)
    s = pl.program_id(2)
    na, nb = _node_pair(rnd, s)
    qa = feats_ref[0, pl.ds(na * NG, NG)].reshape(NG * P, D)
    rb = feats_ref[0, pl.ds(nb * NG, NG)].reshape(NG * P, D)
    dims = (((1,), (1,)), ((), ()))
    sqa = sqb_all_ref[0, pl.ds(na * NG, NG)].reshape(NG * P, 128)
    sqb = sqb_all_ref[0, pl.ds(nb * NG, NG)].reshape(NG * P, 128)
    ncopy = NG * P // 128
    st = jax.lax.dot_general(rb, qa, dims, preferred_element_type=jnp.float32)
    sqbw = jnp.concatenate([sqb] * ncopy, axis=1)
    ma_ref[0, 0] = jnp.min((sqbw - 2.0 * st).reshape(NG, P, NG * P), axis=1)

    @pl.when(rnd < NRN)
    def _cross():
        s2 = jax.lax.dot_general(qa, rb, dims,
                                 preferred_element_type=jnp.float32)
        sqaw = jnp.concatenate([sqa] * ncopy, axis=1)
        mb_ref[0, 0] = jnp.min((sqaw - 2.0 * s2).reshape(NG, P, NG * P), axis=1)


def _k3a_body(ma_ref, mb_ref, sql_ref, scores_ref):
    ma = ma_ref[...]
    mb = mb_ref[...]
    sql = sql_ref[...][:, None]
    big = jnp.float32(3.0e38)
    img = jax.lax.broadcasted_iota(jnp.int32, ma.shape, 3) // P
    rnd = jax.lax.broadcasted_iota(jnp.int32, ma.shape, 1)
    opp = jax.lax.broadcasted_iota(jnp.int32, ma.shape, 2)
    node = img // NG
    side = (node - rnd) % NRN
    is_a = (node == NN - 1) | ((side >= 1) & (side <= NSL - 1))
    cross = jnp.where(is_a, ma, mb)
    diag_ok = (rnd == NRN + node // NSL) & (opp != img % NG)
    vals = jnp.where(rnd < NRN, cross, jnp.where(diag_ok, ma, big))
    cidx = rnd * NG + opp
    total = jnp.zeros((NF, 1, 1, B * P), jnp.float32)
    for _ in range(KSEL):
        mv = jnp.min(vals, axis=(1, 2), keepdims=True)
        total = total + jnp.sqrt(jnp.maximum(mv + sql, 1e-12))
        eq = vals <= mv
        idx = jnp.where(eq, cidx, NG * NRT)
        first = cidx == jnp.min(idx, axis=(1, 2), keepdims=True)
        vals = jnp.where(first, big, vals)
    scores_ref[...] = jnp.mean(total, axis=0)[0, 0][None] * (1.0 / KSEL)


def _k3b_body(scores_ref, m_ref, pix_ref, final_ref):
    g = scores_ref[0]
    final_ref[0, 0] = jnp.full((128,), jnp.max(g), jnp.float32)
    m = m_ref[...]
    a1 = jax.lax.dot_general(m, g, (((1,), (0,)), ((), ())),
                             preferred_element_type=jnp.float32)
    pix_ref[0] = jax.lax.dot_general(a1, m, (((1,), (1,)), ((), ())),
                                     preferred_element_type=jnp.float32)


@jax.jit
def kernel(pixel_values, W_patch, b_patch, W_layers, b_layers):
    patches = pixel_values.reshape(B, 3, PH, PATCH, PW, PATCH)
    patches = patches.transpose(0, 2, 4, 1, 3, 5).reshape(B, P, 3 * PATCH * PATCH)
    cdim = patches.shape[-1]

    feats, sqb, sql = pl.pallas_call(
        _k1_body,
        grid=(B,),
        in_specs=[
            pl.BlockSpec((1, P, cdim), lambda b: (b, 0, 0)),
            pl.BlockSpec((cdim, D), lambda b: (0, 0)),
            pl.BlockSpec((D,), lambda b: (0,)),
            pl.BlockSpec((L, D, D), lambda b: (0, 0, 0)),
            pl.BlockSpec((L, D), lambda b: (0, 0)),
            pl.BlockSpec((P, P), lambda b: (0, 0)),
        ],
        out_specs=[
            pl.BlockSpec((NF, 1, P, D), lambda b: (0, b, 0, 0)),
            pl.BlockSpec((NF, 1, P, 128), lambda b: (0, b, 0, 0)),
            pl.BlockSpec((NF, 1, P), lambda b: (0, 0, b)),
        ],
        out_shape=[
            jax.ShapeDtypeStruct((NF, B, P, D), jnp.bfloat16),
            jax.ShapeDtypeStruct((NF, B, P, 128), jnp.float32),
            jax.ShapeDtypeStruct((NF, 1, B * P), jnp.float32),
        ],
    )(patches, W_patch, b_patch, W_layers, b_layers, jnp.asarray(_POOL))

    ma, mb = pl.pallas_call(
        _k2_body,
        grid=(NF, NRT, NSL),
        in_specs=[
            pl.BlockSpec((1, B, P, D), lambda f, r, s: (f, 0, 0, 0)),
            pl.BlockSpec((1, B, P, 128), lambda f, r, s: (f, 0, 0, 0)),
        ],
        out_specs=[
            pl.BlockSpec((1, 1, NG, NG * P),
                         lambda f, r, s: (f, r, 0, _node_pair(r, s)[0])),
            pl.BlockSpec((1, 1, NG, NG * P),
                         lambda f, r, s: (f, r, 0, _node_pair(r, s)[1])),
        ],
        out_shape=[
            jax.ShapeDtypeStruct((NF, NRT, NG, B * P), jnp.float32),
            jax.ShapeDtypeStruct((NF, NRT, NG, B * P), jnp.float32),
        ],
    )(feats, sqb)

    scores = pl.pallas_call(
        _k3a_body,
        in_specs=[
            pl.BlockSpec((NF, NRT, NG, B * P), lambda: (0, 0, 0, 0)),
            pl.BlockSpec((NF, NRT, NG, B * P), lambda: (0, 0, 0, 0)),
            pl.BlockSpec((NF, 1, B * P), lambda: (0, 0, 0)),
        ],
        out_specs=pl.BlockSpec((1, B * P), lambda: (0, 0)),
        out_shape=jax.ShapeDtypeStruct((1, B * P), jnp.float32),
    )(ma, mb, sql)

    pix, final = pl.pallas_call(
        _k3b_body,
        grid=(B,),
        in_specs=[
            pl.BlockSpec((1, PH, PW), lambda b: (b, 0, 0)),
            pl.BlockSpec((H, PH), lambda b: (0, 0)),
        ],
        out_specs=[
            pl.BlockSpec((1, H, W), lambda b: (b, 0, 0)),
            pl.BlockSpec((1, 1, 128), lambda b: (b, 0, 0)),
        ],
        out_shape=[
            jax.ShapeDtypeStruct((B, H, W), jnp.float32),
            jax.ShapeDtypeStruct((B, 1, 128), jnp.float32),
        ],
    )(scores.reshape(B, PH, PW), jnp.asarray(_RESIZE))

    return final[:, 0, 0], pix


# speedup vs baseline: 20.9503x; 1.0121x over previous
import jax
import jax.numpy as jnp
import numpy as np
from jax.experimental import pallas as pl

B = 16
H = 224
W = 224
PATCH = 14
PH = H // PATCH
PW = W // PATCH
P = PH * PW
D = 1024
L = 2
NF = 4
NG = 4
NN = B // NG
NRN = NN - 1
NSL = NN // 2
NRT = NRN + 2
KSEL = 5


def _pool_matrix() -> np.ndarray:
    a = np.zeros((PH, PH), np.float32)
    for i in range(PH):
        lo, hi = max(0, i - 1), min(PH - 1, i + 1)
        a[i, lo:hi + 1] = 1.0 / (hi - lo + 1)
    return np.kron(a, a).astype(np.float32)


def _resize_matrix() -> np.ndarray:
    scale = H / PH
    out = np.zeros((H, PH), np.float32)
    for i in range(H):
        x = (i + 0.5) / scale - 0.5
        w = np.maximum(0.0, 1.0 - np.abs(x - np.arange(PH)))
        out[i] = w / w.sum()
    return out


_POOL = _pool_matrix()
_RESIZE = _resize_matrix()


def _bf16_dot(x, y, dims):
    return jax.lax.dot_general(
        x.astype(jnp.bfloat16), y.astype(jnp.bfloat16), dims,
        preferred_element_type=jnp.float32)


def _k1_body(patches_ref, wp_ref, bp_ref, wl_ref, bl_ref, pool_ref,
             feats_ref, sqb_ref, sql_ref):
    t = _bf16_dot(patches_ref[...].reshape(NG * P, 3 * PATCH * PATCH),
                  wp_ref[...], (((1,), (0,)), ((), ()))) + bp_ref[...]
    pool = pool_ref[...]
    ones = jnp.ones((D, 128), jnp.bfloat16)

    def emit(i, f):
        feats_ref[i] = f.reshape(NG, P, D).astype(jnp.bfloat16)
        f2 = f * f
        hi = f2.astype(jnp.bfloat16)
        lo = (f2 - hi.astype(jnp.float32)).astype(jnp.bfloat16)
        dims = (((1,), (0,)), ((), ()))
        sq = (jax.lax.dot_general(hi, ones, dims,
                                  preferred_element_type=jnp.float32)
              + jax.lax.dot_general(lo, ones, dims,
                                    preferred_element_type=jnp.float32))
        sqb_ref[i] = sq.reshape(NG, P, 128)
        sql_ref[i, 0] = jnp.sum(f2, axis=1)

    for l in range(L):
        x = _bf16_dot(t, wl_ref[l], (((1,), (0,)), ((), ()))) + bl_ref[l]
        f = jax.nn.gelu(x)
        emit(l, f)
        pooled = jnp.concatenate(
            [_bf16_dot(pool, f[g * P:(g + 1) * P], (((1,), (0,)), ((), ())))
             for g in range(NG)], axis=0)
        emit(2 + l, pooled)


def _node_pair(rnd, s):
    diag_n = (rnd - NRN) * NSL + s
    na = jnp.where(rnd >= NRN, diag_n,
                   jnp.where(s == 0, NN - 1, (rnd + s) % NRN))
    nb = jnp.where(rnd >= NRN, diag_n, (rnd - s) % NRN)
    return na, nb


def _k2_body(feats_ref, sqb_all_ref, ma_ref, mb_ref):
    rnd = pl.program_id(1)
    s = pl.program_id(2)
    na, nb = _node_pair(rnd, s)
    qa = feats_ref[0, pl.ds(na * NG, NG)].reshape(NG * P, D)
    rb = feats_ref[0, pl.ds(nb * NG, NG)].reshape(NG * P, D)
    dims = (((1,), (1,)), ((), ()))
    sqa = sqb_all_ref[0, pl.ds(na * NG, NG)].reshape(NG * P, 128)
    sqb = sqb_all_ref[0, pl.ds(nb * NG, NG)].reshape(NG * P, 128)
    ncopy = NG * P // 128
    st = jax.lax.dot_general(rb, qa, dims, preferred_element_type=jnp.float32)
    sqbw = jnp.concatenate([sqb] * ncopy, axis=1)
    ma_ref[0, 0] = jnp.min((sqbw - 2.0 * st).reshape(NG, P, NG * P), axis=1)

    @pl.when(rnd < NRN)
    def _cross():
        s2 = jax.lax.dot_general(qa, rb, dims,
                                 preferred_element_type=jnp.float32)
        sqaw = jnp.concatenate([sqa] * ncopy, axis=1)
        mb_ref[0, 0] = jnp.min((sqaw - 2.0 * s2).reshape(NG, P, NG * P), axis=1)


def _k3a_body(ma_ref, mb_ref, sql_ref, scores_ref):
    ma = ma_ref[...]
    mb = mb_ref[...]
    sql = sql_ref[...][:, None]
    big = jnp.float32(3.0e38)
    img = jax.lax.broadcasted_iota(jnp.int32, ma.shape, 3) // P
    rnd = jax.lax.broadcasted_iota(jnp.int32, ma.shape, 1)
    opp = jax.lax.broadcasted_iota(jnp.int32, ma.shape, 2)
    node = img // NG
    side = (node - rnd) % NRN
    is_a = (node == NN - 1) | ((side >= 1) & (side <= NSL - 1))
    cross = jnp.where(is_a, ma, mb)
    diag_ok = (rnd == NRN + node // NSL) & (opp != img % NG)
    vals = jnp.where(rnd < NRN, cross, jnp.where(diag_ok, ma, big))
    cidx = rnd * NG + opp
    total = jnp.zeros((NF, 1, 1, B * P), jnp.float32)
    for _ in range(KSEL):
        mv = jnp.min(vals, axis=(1, 2), keepdims=True)
        total = total + jnp.sqrt(jnp.maximum(mv + sql, 1e-12))
        eq = vals <= mv
        idx = jnp.where(eq, cidx, NG * NRT)
        first = cidx == jnp.min(idx, axis=(1, 2), keepdims=True)
        vals = jnp.where(first, big, vals)
    scores_ref[...] = jnp.mean(total, axis=0)[0, 0][None] * (1.0 / KSEL)


def _k3b_body(scores_ref, m_ref, pix_ref, final_ref):
    g = scores_ref[0]
    final_ref[0, 0] = jnp.full((128,), jnp.max(g), jnp.float32)
    m = m_ref[...]
    a1 = jax.lax.dot_general(m, g, (((1,), (0,)), ((), ())),
                             preferred_element_type=jnp.float32)
    pix_ref[0] = jax.lax.dot_general(a1, m, (((1,), (1,)), ((), ())),
                                     preferred_element_type=jnp.float32)


@jax.jit
def kernel(pixel_values, W_patch, b_patch, W_layers, b_layers):
    patches = pixel_values.reshape(B, 3, PH, PATCH, PW, PATCH)
    patches = patches.transpose(0, 2, 4, 1, 3, 5).reshape(B, P, 3 * PATCH * PATCH)
    cdim = patches.shape[-1]

    feats, sqb, sql = pl.pallas_call(
        _k1_body,
        grid=(B // NG,),
        in_specs=[
            pl.BlockSpec((NG, P, cdim), lambda b: (b, 0, 0)),
            pl.BlockSpec((cdim, D), lambda b: (0, 0)),
            pl.BlockSpec((D,), lambda b: (0,)),
            pl.BlockSpec((L, D, D), lambda b: (0, 0, 0)),
            pl.BlockSpec((L, D), lambda b: (0, 0)),
            pl.BlockSpec((P, P), lambda b: (0, 0)),
        ],
        out_specs=[
            pl.BlockSpec((NF, NG, P, D), lambda b: (0, b, 0, 0)),
            pl.BlockSpec((NF, NG, P, 128), lambda b: (0, b, 0, 0)),
            pl.BlockSpec((NF, 1, NG * P), lambda b: (0, 0, b)),
        ],
        out_shape=[
            jax.ShapeDtypeStruct((NF, B, P, D), jnp.bfloat16),
            jax.ShapeDtypeStruct((NF, B, P, 128), jnp.float32),
            jax.ShapeDtypeStruct((NF, 1, B * P), jnp.float32),
        ],
    )(patches, W_patch, b_patch, W_layers, b_layers, jnp.asarray(_POOL))

    ma, mb = pl.pallas_call(
        _k2_body,
        grid=(NF, NRT, NSL),
        in_specs=[
            pl.BlockSpec((1, B, P, D), lambda f, r, s: (f, 0, 0, 0)),
            pl.BlockSpec((1, B, P, 128), lambda f, r, s: (f, 0, 0, 0)),
        ],
        out_specs=[
            pl.BlockSpec((1, 1, NG, NG * P),
                         lambda f, r, s: (f, r, 0, _node_pair(r, s)[0])),
            pl.BlockSpec((1, 1, NG, NG * P),
                         lambda f, r, s: (f, r, 0, _node_pair(r, s)[1])),
        ],
        out_shape=[
            jax.ShapeDtypeStruct((NF, NRT, NG, B * P), jnp.float32),
            jax.ShapeDtypeStruct((NF, NRT, NG, B * P), jnp.float32),
        ],
    )(feats, sqb)

    scores = pl.pallas_call(
        _k3a_body,
        in_specs=[
            pl.BlockSpec((NF, NRT, NG, B * P), lambda: (0, 0, 0, 0)),
            pl.BlockSpec((NF, NRT, NG, B * P), lambda: (0, 0, 0, 0)),
            pl.BlockSpec((NF, 1, B * P), lambda: (0, 0, 0)),
        ],
        out_specs=pl.BlockSpec((1, B * P), lambda: (0, 0)),
        out_shape=jax.ShapeDtypeStruct((1, B * P), jnp.float32),
    )(ma, mb, sql)

    pix, final = pl.pallas_call(
        _k3b_body,
        grid=(B,),
        in_specs=[
            pl.BlockSpec((1, PH, PW), lambda b: (b, 0, 0)),
            pl.BlockSpec((H, PH), lambda b: (0, 0)),
        ],
        out_specs=[
            pl.BlockSpec((1, H, W), lambda b: (b, 0, 0)),
            pl.BlockSpec((1, 1, 128), lambda b: (b, 0, 0)),
        ],
        out_shape=[
            jax.ShapeDtypeStruct((B, H, W), jnp.float32),
            jax.ShapeDtypeStruct((B, 1, 128), jnp.float32),
        ],
    )(scores.reshape(B, PH, PW), jnp.asarray(_RESIZE))

    return final[:, 0, 0], pix


# speedup vs baseline: 21.6116x; 1.0316x over previous
import jax
import jax.numpy as jnp
import numpy as np
from jax.experimental import pallas as pl

B = 16
H = 224
W = 224
PATCH = 14
PH = H // PATCH
PW = W // PATCH
P = PH * PW
D = 1024
L = 2
NF = 4
NG = 4
NN = B // NG
NRN = NN - 1
NSL = NN // 2
NRT = NRN + 2
KSEL = 5


def _pool_matrix() -> np.ndarray:
    a = np.zeros((PH, PH), np.float32)
    for i in range(PH):
        lo, hi = max(0, i - 1), min(PH - 1, i + 1)
        a[i, lo:hi + 1] = 1.0 / (hi - lo + 1)
    return np.kron(a, a).astype(np.float32)


def _resize_matrix() -> np.ndarray:
    scale = H / PH
    out = np.zeros((H, PH), np.float32)
    for i in range(H):
        x = (i + 0.5) / scale - 0.5
        w = np.maximum(0.0, 1.0 - np.abs(x - np.arange(PH)))
        out[i] = w / w.sum()
    return out


_POOL = _pool_matrix()
_RESIZE = _resize_matrix()
_A1 = _RESIZE[:, np.arange(P) // PW].astype(np.float32)
_A2 = _RESIZE[:, np.arange(P) % PW].astype(np.float32).T.copy()


def _bf16_dot(x, y, dims):
    return jax.lax.dot_general(
        x.astype(jnp.bfloat16), y.astype(jnp.bfloat16), dims,
        preferred_element_type=jnp.float32)


def _k1_body(patches_ref, wp_ref, bp_ref, wl_ref, bl_ref, pool_ref,
             feats_ref, sqb_ref, sql_ref):
    t = _bf16_dot(patches_ref[...].reshape(NG * P, 3 * PATCH * PATCH),
                  wp_ref[...], (((1,), (0,)), ((), ()))) + bp_ref[...]
    pool = pool_ref[...]
    ones = jnp.ones((D, 128), jnp.bfloat16)

    def emit(i, f):
        feats_ref[i] = f.reshape(NG, P, D).astype(jnp.bfloat16)
        f2 = f * f
        hi = f2.astype(jnp.bfloat16)
        lo = (f2 - hi.astype(jnp.float32)).astype(jnp.bfloat16)
        dims = (((1,), (0,)), ((), ()))
        sq = (jax.lax.dot_general(hi, ones, dims,
                                  preferred_element_type=jnp.float32)
              + jax.lax.dot_general(lo, ones, dims,
                                    preferred_element_type=jnp.float32))
        sqb_ref[i] = sq.reshape(NG, P, 128)
        sql_ref[i, 0] = jnp.sum(f2, axis=1)

    for l in range(L):
        x = _bf16_dot(t, wl_ref[l], (((1,), (0,)), ((), ()))) + bl_ref[l]
        f = jax.nn.gelu(x)
        emit(l, f)
        pooled = jnp.concatenate(
            [_bf16_dot(pool, f[g * P:(g + 1) * P], (((1,), (0,)), ((), ())))
             for g in range(NG)], axis=0)
        emit(2 + l, pooled)


def _node_pair(rnd, s):
    diag_n = (rnd - NRN) * NSL + s
    na = jnp.where(rnd >= NRN, diag_n,
                   jnp.where(s == 0, NN - 1, (rnd + s) % NRN))
    nb = jnp.where(rnd >= NRN, diag_n, (rnd - s) % NRN)
    return na, nb


def _k2_body(feats_ref, sqb_all_ref, ma_ref, mb_ref):
    rnd = pl.program_id(1)
    s = pl.program_id(2)
    na, nb = _node_pair(rnd, s)
    qa = feats_ref[0, pl.ds(na * NG, NG)].reshape(NG * P, D)
    rb = feats_ref[0, pl.ds(nb * NG, NG)].reshape(NG * P, D)
    dims = (((1,), (1,)), ((), ()))
    sqa = sqb_all_ref[0, pl.ds(na * NG, NG)].reshape(NG * P, 128)
    sqb = sqb_all_ref[0, pl.ds(nb * NG, NG)].reshape(NG * P, 128)
    ncopy = NG * P // 128
    st = jax.lax.dot_general(rb, qa, dims, preferred_element_type=jnp.float32)
    sqbw = jnp.concatenate([sqb] * ncopy, axis=1)
    ma_ref[0, 0] = jnp.min((sqbw - 2.0 * st).reshape(NG, P, NG * P), axis=1)

    @pl.when(rnd < NRN)
    def _cross():
        s2 = jax.lax.dot_general(qa, rb, dims,
                                 preferred_element_type=jnp.float32)
        sqaw = jnp.concatenate([sqa] * ncopy, axis=1)
        mb_ref[0, 0] = jnp.min((sqaw - 2.0 * s2).reshape(NG, P, NG * P), axis=1)


def _k3a_body(ma_ref, mb_ref, sql_ref, a1_ref, a2_ref, pix_ref, final_ref):
    ma = ma_ref[...]
    mb = mb_ref[...]
    sql = sql_ref[...][:, None]
    big = jnp.float32(3.0e38)
    img = jax.lax.broadcasted_iota(jnp.int32, ma.shape, 3) // P
    rnd = jax.lax.broadcasted_iota(jnp.int32, ma.shape, 1)
    opp = jax.lax.broadcasted_iota(jnp.int32, ma.shape, 2)
    node = img // NG
    side = (node - rnd) % NRN
    is_a = (node == NN - 1) | ((side >= 1) & (side <= NSL - 1))
    cross = jnp.where(is_a, ma, mb)
    diag_ok = (rnd == NRN + node // NSL) & (opp != img % NG)
    vals = jnp.where(rnd < NRN, cross, jnp.where(diag_ok, ma, big))
    cidx = rnd * NG + opp
    total = jnp.zeros((NF, 1, 1, B * P), jnp.float32)
    for _ in range(KSEL):
        mv = jnp.min(vals, axis=(1, 2), keepdims=True)
        total = total + jnp.sqrt(jnp.maximum(mv + sql, 1e-12))
        eq = vals <= mv
        idx = jnp.where(eq, cidx, NG * NRT)
        first = cidx == jnp.min(idx, axis=(1, 2), keepdims=True)
        vals = jnp.where(first, big, vals)
    sc = jnp.mean(total, axis=0)[0, 0] * (1.0 / KSEL)
    a1 = a1_ref[...]
    a2 = a2_ref[...]
    final_ref[0] = jnp.stack([jnp.max(sc[b * P:(b + 1) * P]) for b in range(B)])
    for b in range(B):
        x = (a1 * sc[b * P:(b + 1) * P][None, :]).astype(jnp.bfloat16)
        pix_ref[b] = jax.lax.dot_general(
            x, a2.astype(jnp.bfloat16), (((1,), (0,)), ((), ())),
            preferred_element_type=jnp.float32)


@jax.jit
def kernel(pixel_values, W_patch, b_patch, W_layers, b_layers):
    patches = pixel_values.reshape(B, 3, PH, PATCH, PW, PATCH)
    patches = patches.transpose(0, 2, 4, 1, 3, 5).reshape(B, P, 3 * PATCH * PATCH)
    cdim = patches.shape[-1]

    feats, sqb, sql = pl.pallas_call(
        _k1_body,
        grid=(B // NG,),
        in_specs=[
            pl.BlockSpec((NG, P, cdim), lambda b: (b, 0, 0)),
            pl.BlockSpec((cdim, D), lambda b: (0, 0)),
            pl.BlockSpec((D,), lambda b: (0,)),
            pl.BlockSpec((L, D, D), lambda b: (0, 0, 0)),
            pl.BlockSpec((L, D), lambda b: (0, 0)),
            pl.BlockSpec((P, P), lambda b: (0, 0)),
        ],
        out_specs=[
            pl.BlockSpec((NF, NG, P, D), lambda b: (0, b, 0, 0)),
            pl.BlockSpec((NF, NG, P, 128), lambda b: (0, b, 0, 0)),
            pl.BlockSpec((NF, 1, NG * P), lambda b: (0, 0, b)),
        ],
        out_shape=[
            jax.ShapeDtypeStruct((NF, B, P, D), jnp.bfloat16),
            jax.ShapeDtypeStruct((NF, B, P, 128), jnp.float32),
            jax.ShapeDtypeStruct((NF, 1, B * P), jnp.float32),
        ],
    )(patches, W_patch, b_patch, W_layers, b_layers, jnp.asarray(_POOL))

    ma, mb = pl.pallas_call(
        _k2_body,
        grid=(NF, NRT, NSL),
        in_specs=[
            pl.BlockSpec((1, B, P, D), lambda f, r, s: (f, 0, 0, 0)),
            pl.BlockSpec((1, B, P, 128), lambda f, r, s: (f, 0, 0, 0)),
        ],
        out_specs=[
            pl.BlockSpec((1, 1, NG, NG * P),
                         lambda f, r, s: (f, r, 0, _node_pair(r, s)[0])),
            pl.BlockSpec((1, 1, NG, NG * P),
                         lambda f, r, s: (f, r, 0, _node_pair(r, s)[1])),
        ],
        out_shape=[
            jax.ShapeDtypeStruct((NF, NRT, NG, B * P), jnp.float32),
            jax.ShapeDtypeStruct((NF, NRT, NG, B * P), jnp.float32),
        ],
    )(feats, sqb)

    pix, final = pl.pallas_call(
        _k3a_body,
        in_specs=[
            pl.BlockSpec((NF, NRT, NG, B * P), lambda: (0, 0, 0, 0)),
            pl.BlockSpec((NF, NRT, NG, B * P), lambda: (0, 0, 0, 0)),
            pl.BlockSpec((NF, 1, B * P), lambda: (0, 0, 0)),
            pl.BlockSpec((H, P), lambda: (0, 0)),
            pl.BlockSpec((P, H), lambda: (0, 0)),
        ],
        out_specs=[
            pl.BlockSpec((B, H, W), lambda: (0, 0, 0)),
            pl.BlockSpec((1, B), lambda: (0, 0)),
        ],
        out_shape=[
            jax.ShapeDtypeStruct((B, H, W), jnp.float32),
            jax.ShapeDtypeStruct((1, B), jnp.float32),
        ],
    )(ma, mb, sql, jnp.asarray(_A1), jnp.asarray(_A2))

    return final[0], pix
